# baseline (speedup 1.0000x reference)
import functools

import jax
import jax.numpy as jnp
from jax import lax
from jax.experimental import pallas as pl
from jax.experimental.pallas import tpu as pltpu
from jax.experimental.pallas import tpu_sc as plsc


_CHUNK_ROWS = 16
_NBUF = 3


def _sc_body(pe_hbm, out_hbm, buf, in_sems, out_sems):
    nw = 32
    rows_per_w = pe_hbm.shape[0] // nw
    n = rows_per_w // _CHUNK_ROWS
    wid = lax.axis_index("s") * 2 + lax.axis_index("c")
    base = wid * rows_per_w

    def in_copy(i, slot):
        return pltpu.make_async_copy(
            pe_hbm.at[pl.ds(base + i * _CHUNK_ROWS, _CHUNK_ROWS)],
            buf.at[slot], in_sems.at[slot])

    def out_copy(i, slot):
        return pltpu.make_async_copy(
            buf.at[slot],
            out_hbm.at[pl.ds(base + i * _CHUNK_ROWS, _CHUNK_ROWS)],
            out_sems.at[slot])

    for i in range(min(_NBUF, n)):
        in_copy(i, i % _NBUF).start()
    for i in range(n):
        slot = i % _NBUF
        in_copy(i, slot).wait()
        out_copy(i, slot).start()
        nxt = i + _NBUF
        if nxt < n:
            out_copy(nxt - _NBUF, slot).wait()
            in_copy(nxt, slot).start()
    for i in range(max(n - _NBUF, 0), n):
        out_copy(i, i % _NBUF).wait()


def kernel(x, pe_weight):
    seq_len = x.shape[1]
    hidden = pe_weight.shape[1]
    k = functools.partial(
        pl.kernel,
        mesh=plsc.VectorSubcoreMesh(core_axis_name="c", subcore_axis_name="s"),
        out_type=jax.ShapeDtypeStruct((seq_len, hidden), pe_weight.dtype),
        scratch_types=[
            pltpu.VMEM((_NBUF, _CHUNK_ROWS, hidden), pe_weight.dtype),
            pltpu.SemaphoreType.DMA((_NBUF,)),
            pltpu.SemaphoreType.DMA((_NBUF,)),
        ],
    )(_sc_body)
    out = k(pe_weight)
    return out[None]


# speedup vs baseline: 2.0460x; 2.0460x over previous
import jax
import jax.numpy as jnp
from jax.experimental import pallas as pl
from jax.experimental.pallas import tpu as pltpu


_CHUNK_ROWS = 4096


def _copy_body(src_hbm, dst_hbm, buf, in_sems, out_sems):
    rows = src_hbm.shape[0]
    n = rows // _CHUNK_ROWS

    def in_copy(i):
        return pltpu.make_async_copy(
            src_hbm.at[pl.ds(i * _CHUNK_ROWS, _CHUNK_ROWS)], buf.at[i],
            in_sems.at[i])

    def out_copy(i):
        return pltpu.make_async_copy(
            buf.at[i], dst_hbm.at[0, pl.ds(i * _CHUNK_ROWS, _CHUNK_ROWS)],
            out_sems.at[i])

    for i in range(n):
        in_copy(i).start()
    for i in range(n):
        in_copy(i).wait()
        out_copy(i).start()
    for i in range(n):
        out_copy(i).wait()


def kernel(x, pe_weight):
    seq_len = x.shape[1]
    hidden = pe_weight.shape[1]
    n = seq_len // _CHUNK_ROWS
    return pl.pallas_call(
        _copy_body,
        out_shape=jax.ShapeDtypeStruct((1, seq_len, hidden), pe_weight.dtype),
        in_specs=[pl.BlockSpec(memory_space=pl.ANY)],
        out_specs=pl.BlockSpec(memory_space=pl.ANY),
        scratch_shapes=[
            pltpu.VMEM((n, _CHUNK_ROWS, hidden), pe_weight.dtype),
            pltpu.SemaphoreType.DMA((n,)),
            pltpu.SemaphoreType.DMA((n,)),
        ],
    )(pe_weight)
